# Initial kernel scaffold; baseline (speedup 1.0000x reference)
import functools

import jax
import jax.numpy as jnp
from jax import lax
from jax.experimental import pallas as pl
from jax.experimental.pallas import tpu as pltpu
from jax.experimental.pallas import tpu_sc as plsc

DIM = 128
NUM_X_SHIFTS = 15
NUM_EDGE_EMB = 225
NUM_BRICKS = 1024
N_LEGO = 10000
N_EDGE = 320000
N_POINT = 50000

NC = 2
NS = 16
NW = NC * NS
L = 16

CH = 128
NBLK = N_EDGE // CH
NITER = -(-NBLK // NW)


def _edge_gather_body(shifts_hbm, table_hbm, out_hbm, shifts_v, idx_v, rows_v, sem):
    wid = lax.axis_index("s") * NC + lax.axis_index("c")

    def body(t, carry):
        g = t * NW + wid

        @pl.when(g < NBLK)
        def _():
            base = g * CH
            pltpu.sync_copy(shifts_hbm.at[pl.ds(base, CH)], shifts_v)
            lane = lax.iota(jnp.int32, L)
            zero = jnp.zeros((L,), jnp.int32)
            one = jnp.ones((L,), jnp.int32)
            for j in range(CH // L):
                r = lane + (j * L)
                s0 = plsc.load_gather(shifts_v, [r, zero])
                s1 = plsc.load_gather(shifts_v, [r, one])
                idx_v[pl.ds(j * L, L)] = s0 * NUM_X_SHIFTS + s1
            pltpu.async_copy(table_hbm.at[idx_v], rows_v, sem).wait()
            pltpu.sync_copy(rows_v, out_hbm.at[pl.ds(base, CH)])

        return carry

    lax.fori_loop(0, NITER, body, 0)


_edge_gather = functools.partial(
    pl.kernel,
    out_type=jax.ShapeDtypeStruct((N_EDGE, DIM), jnp.float32),
    mesh=plsc.VectorSubcoreMesh(
        core_axis_name="c", subcore_axis_name="s", num_cores=NC, num_subcores=NS
    ),
    scratch_types=[
        pltpu.VMEM((CH, 2), jnp.int32),
        pltpu.VMEM((CH,), jnp.int32),
        pltpu.VMEM((CH, DIM), jnp.float32),
        pltpu.SemaphoreType.DMA,
    ],
)(_edge_gather_body)


LEGO_BLK = 1000
POINT_BLK = 5000
GRID = N_LEGO // LEGO_BLK


def _dense_body(ids_ref, lego_pos_ref, point_pos_ref, brick_ref, wpos_ref,
                bpos_ref, wpoint_ref, bpoint_ref, lego_out_ref, point_out_ref):
    ids = ids_ref[0, 0, :]
    onehot = (ids[:, None] == lax.broadcasted_iota(
        jnp.int32, (LEGO_BLK, NUM_BRICKS), 1)).astype(jnp.bfloat16)
    brick = jnp.dot(onehot, brick_ref[...], preferred_element_type=jnp.float32)
    proj = jnp.dot(lego_pos_ref[...], wpos_ref[...],
                   preferred_element_type=jnp.float32)
    lego_out_ref[...] = brick + proj + bpos_ref[...]
    point_out_ref[...] = jnp.dot(point_pos_ref[...], wpoint_ref[...],
                                 preferred_element_type=jnp.float32) + bpoint_ref[...]


def _dense(ids3, lego_pos, point_pos, brick_bf16, w_pos, b_pos, w_point, b_point):
    return pl.pallas_call(
        _dense_body,
        grid=(GRID,),
        in_specs=[
            pl.BlockSpec((1, 1, LEGO_BLK), lambda i: (i, 0, 0)),
            pl.BlockSpec((LEGO_BLK, 8), lambda i: (i, 0)),
            pl.BlockSpec((POINT_BLK, 8), lambda i: (i, 0)),
            pl.BlockSpec((NUM_BRICKS, DIM), lambda i: (0, 0)),
            pl.BlockSpec((8, DIM), lambda i: (0, 0)),
            pl.BlockSpec((1, DIM), lambda i: (0, 0)),
            pl.BlockSpec((8, DIM), lambda i: (0, 0)),
            pl.BlockSpec((1, DIM), lambda i: (0, 0)),
        ],
        out_specs=[
            pl.BlockSpec((LEGO_BLK, DIM), lambda i: (i, 0)),
            pl.BlockSpec((POINT_BLK, DIM), lambda i: (i, 0)),
        ],
        out_shape=[
            jax.ShapeDtypeStruct((N_LEGO, DIM), jnp.float32),
            jax.ShapeDtypeStruct((N_POINT, DIM), jnp.float32),
        ],
    )(ids3, lego_pos, point_pos, brick_bf16, w_pos, b_pos, w_point, b_point)


def kernel(lego_brick_ids, lego_pos, edge_shifts, point_pos,
           brick_table, edge_table, W_pos, b_pos, W_point, b_point):
    edge_attr = _edge_gather(edge_shifts.astype(jnp.int32), edge_table)

    ids3 = lego_brick_ids.astype(jnp.int32).reshape(GRID, 1, LEGO_BLK)
    lego_pos8 = jnp.pad(lego_pos, ((0, 0), (0, 2)))
    point_pos8 = jnp.pad(point_pos, ((0, 0), (0, 5)))
    wpos8 = jnp.pad(W_pos, ((0, 2), (0, 0)))
    wpoint8 = jnp.pad(W_point, ((0, 5), (0, 0)))
    lego_x, point_x = _dense(
        ids3, lego_pos8, point_pos8, brick_table.astype(jnp.bfloat16),
        wpos8, b_pos.reshape(1, DIM), wpoint8, b_point.reshape(1, DIM))
    return (lego_x, edge_attr, point_x)


# speedup vs baseline: 1.3924x; 1.3924x over previous
import functools

import jax
import jax.numpy as jnp
from jax import lax
from jax.experimental import pallas as pl
from jax.experimental.pallas import tpu as pltpu
from jax.experimental.pallas import tpu_sc as plsc

DIM = 128
NUM_X_SHIFTS = 15
NUM_EDGE_EMB = 225
NUM_BRICKS = 1024
N_LEGO = 10000
N_EDGE = 320000
N_POINT = 50000

NC = 2
NS = 16
NW = NC * NS
L = 16

CH = 128
NBLK = N_EDGE // CH
NITER = -(-NBLK // NW)


def _edge_gather_body(shifts_hbm, table_hbm, out_hbm, shifts_v, idx_v, rows_v, sem):
    wid = lax.axis_index("s") * NC + lax.axis_index("c")

    def body(t, carry):
        g = t * NW + wid

        @pl.when(g < NBLK)
        def _():
            base = g * CH
            pltpu.sync_copy(shifts_hbm.at[pl.ds(base * 2, CH * 2)], shifts_v)
            lane = lax.iota(jnp.int32, L)
            for j in range(CH // L):
                e2 = (lane + (j * L)) * 2
                s0 = plsc.load_gather(shifts_v, [e2])
                s1 = plsc.load_gather(shifts_v, [e2 + 1])
                idx_v[pl.ds(j * L, L)] = s0 * NUM_X_SHIFTS + s1
            pltpu.async_copy(table_hbm.at[idx_v], rows_v, sem).wait()
            pltpu.sync_copy(rows_v, out_hbm.at[pl.ds(base, CH)])

        return carry

    lax.fori_loop(0, NITER, body, 0)


_edge_gather = functools.partial(
    pl.kernel,
    out_type=jax.ShapeDtypeStruct((N_EDGE, DIM), jnp.float32),
    mesh=plsc.VectorSubcoreMesh(
        core_axis_name="c", subcore_axis_name="s", num_cores=NC, num_subcores=NS
    ),
    scratch_types=[
        pltpu.VMEM((CH * 2,), jnp.int32),
        pltpu.VMEM((CH,), jnp.int32),
        pltpu.VMEM((CH, DIM), jnp.float32),
        pltpu.SemaphoreType.DMA,
    ],
    compiler_params=pltpu.CompilerParams(needs_layout_passes=False),
)(_edge_gather_body)


LEGO_BLK = 1000
POINT_BLK = 5000
GRID = N_LEGO // LEGO_BLK


def _dense_body(ids_ref, lego_pos_ref, point_pos_ref, brick_ref, wpos_ref,
                bpos_ref, wpoint_ref, bpoint_ref, lego_out_ref, point_out_ref):
    ids = ids_ref[0, 0, :]
    onehot = (ids[:, None] == lax.broadcasted_iota(
        jnp.int32, (LEGO_BLK, NUM_BRICKS), 1)).astype(jnp.bfloat16)
    brick = jnp.dot(onehot, brick_ref[...], preferred_element_type=jnp.float32)
    proj = jnp.dot(lego_pos_ref[...], wpos_ref[...],
                   preferred_element_type=jnp.float32)
    lego_out_ref[...] = brick + proj + bpos_ref[...]
    point_out_ref[...] = jnp.dot(point_pos_ref[...], wpoint_ref[...],
                                 preferred_element_type=jnp.float32) + bpoint_ref[...]


def _dense(ids3, lego_pos, point_pos, brick_bf16, w_pos, b_pos, w_point, b_point):
    return pl.pallas_call(
        _dense_body,
        grid=(GRID,),
        in_specs=[
            pl.BlockSpec((1, 1, LEGO_BLK), lambda i: (i, 0, 0)),
            pl.BlockSpec((LEGO_BLK, 8), lambda i: (i, 0)),
            pl.BlockSpec((POINT_BLK, 8), lambda i: (i, 0)),
            pl.BlockSpec((NUM_BRICKS, DIM), lambda i: (0, 0)),
            pl.BlockSpec((8, DIM), lambda i: (0, 0)),
            pl.BlockSpec((1, DIM), lambda i: (0, 0)),
            pl.BlockSpec((8, DIM), lambda i: (0, 0)),
            pl.BlockSpec((1, DIM), lambda i: (0, 0)),
        ],
        out_specs=[
            pl.BlockSpec((LEGO_BLK, DIM), lambda i: (i, 0)),
            pl.BlockSpec((POINT_BLK, DIM), lambda i: (i, 0)),
        ],
        out_shape=[
            jax.ShapeDtypeStruct((N_LEGO, DIM), jnp.float32),
            jax.ShapeDtypeStruct((N_POINT, DIM), jnp.float32),
        ],
    )(ids3, lego_pos, point_pos, brick_bf16, w_pos, b_pos, w_point, b_point)


def kernel(lego_brick_ids, lego_pos, edge_shifts, point_pos,
           brick_table, edge_table, W_pos, b_pos, W_point, b_point):
    edge_attr = _edge_gather(
        edge_shifts.astype(jnp.int32).reshape(N_EDGE * 2), edge_table)

    ids3 = lego_brick_ids.astype(jnp.int32).reshape(GRID, 1, LEGO_BLK)
    lego_pos8 = jnp.pad(lego_pos, ((0, 0), (0, 2)))
    point_pos8 = jnp.pad(point_pos, ((0, 0), (0, 5)))
    wpos8 = jnp.pad(W_pos, ((0, 2), (0, 0)))
    wpoint8 = jnp.pad(W_point, ((0, 5), (0, 0)))
    lego_x, point_x = _dense(
        ids3, lego_pos8, point_pos8, brick_table.astype(jnp.bfloat16),
        wpos8, b_pos.reshape(1, DIM), wpoint8, b_point.reshape(1, DIM))
    return (lego_x, edge_attr, point_x)


# speedup vs baseline: 2.6438x; 1.8987x over previous
import functools

import jax
import jax.numpy as jnp
from jax import lax
from jax.experimental import pallas as pl
from jax.experimental.pallas import tpu as pltpu
from jax.experimental.pallas import tpu_sc as plsc

DIM = 128
NUM_X_SHIFTS = 15
NUM_EDGE_EMB = 225
NUM_BRICKS = 1024
N_LEGO = 10000
N_EDGE = 320000
N_POINT = 50000

NC = 2
NS = 16
NW = NC * NS
L = 16

CH = 128
CH2 = 2 * CH
NBLK = N_EDGE // CH
T_MAIN = NBLK // NW
REM = NBLK - T_MAIN * NW
NBUF = 4


def _edge_gather_body(shifts_hbm, table_hbm, out_hbm, table_v, *scratch):
    shifts_v = scratch[0:NBUF]
    idx_v = scratch[NBUF:2 * NBUF]
    rows_v = scratch[2 * NBUF:3 * NBUF]
    ssem = scratch[3 * NBUF:4 * NBUF]
    gsem = scratch[4 * NBUF:5 * NBUF]
    osem = scratch[5 * NBUF:6 * NBUF]
    wid = lax.axis_index("s") * NC + lax.axis_index("c")
    lane = lax.iota(jnp.int32, L)

    @pl.when(lax.axis_index("s") == 0)
    def _():
        pltpu.sync_copy(table_hbm, table_v)

    plsc.subcore_barrier()

    def start_shifts(g, b):
        pltpu.async_copy(shifts_hbm.at[pl.ds(g * CH2, CH2)], shifts_v[b],
                         ssem[b])

    def wait_shifts(g, b):
        pltpu.make_async_copy(shifts_hbm.at[pl.ds(g * CH2, CH2)],
                              shifts_v[b], ssem[b]).wait()

    def compute_idx(b):
        sv = shifts_v[b]
        iv = idx_v[b]
        for j in range(CH // L):
            e2 = (lane + (j * L)) * 2
            s0 = plsc.load_gather(sv, [e2])
            s1 = plsc.load_gather(sv, [e2 + 1])
            iv[pl.ds(j * L, L)] = s0 * NUM_X_SHIFTS + s1

    def start_gather(b):
        pltpu.async_copy(table_v.at[idx_v[b]], rows_v[b], gsem[b])

    def wait_gather(b):
        pltpu.make_async_copy(table_v.at[idx_v[b]], rows_v[b],
                              gsem[b]).wait()

    def start_out(g, b):
        pltpu.async_copy(rows_v[b], out_hbm.at[pl.ds(g * CH, CH)], osem[b])

    def wait_out(g, b):
        pltpu.make_async_copy(rows_v[b], out_hbm.at[pl.ds(g * CH, CH)],
                              osem[b]).wait()

    def g_of(t):
        return t * NW + wid

    for b in range(NBUF):
        start_shifts(g_of(b), b)
    for t in range(NBUF):
        b = t % NBUF
        wait_shifts(g_of(t), b)
        compute_idx(b)
        start_gather(b)
        start_shifts(g_of(t + NBUF), b)
        if t >= 1:
            wait_gather((t - 1) % NBUF)
            start_out(g_of(t - 1), (t - 1) % NBUF)

    def steady(k, carry):
        for b in range(NBUF):
            t = k * NBUF + b
            wait_shifts(g_of(t), b)
            compute_idx(b)
            wait_out(g_of(t - NBUF), b)
            start_gather(b)
            start_shifts(g_of(t + NBUF), b)
            b1 = (b - 1) % NBUF
            wait_gather(b1)
            start_out(g_of(t - 1), b1)
        return carry

    lax.fori_loop(1, T_MAIN // NBUF - 1, steady, 0)

    for t in range(T_MAIN - 6, T_MAIN):
        b = t % NBUF
        wait_shifts(g_of(t), b)
        compute_idx(b)
        wait_out(g_of(t - NBUF), b)
        start_gather(b)
        if t + NBUF < T_MAIN:
            start_shifts(g_of(t + NBUF), b)
        b1 = (b - 1) % NBUF
        wait_gather(b1)
        start_out(g_of(t - 1), b1)
    bl = (T_MAIN - 1) % NBUF
    wait_gather(bl)
    start_out(g_of(T_MAIN - 1), bl)
    for t in range(T_MAIN - NBUF, T_MAIN):
        wait_out(g_of(t), t % NBUF)

    @pl.when(wid < REM)
    def _():
        g = T_MAIN * NW + wid
        pltpu.sync_copy(shifts_hbm.at[pl.ds(g * CH2, CH2)], shifts_v[0])
        compute_idx(0)
        pltpu.async_copy(table_v.at[idx_v[0]], rows_v[0], gsem[0]).wait()
        pltpu.sync_copy(rows_v[0], out_hbm.at[pl.ds(g * CH, CH)])


_edge_gather = functools.partial(
    pl.kernel,
    out_type=jax.ShapeDtypeStruct((N_EDGE, DIM), jnp.float32),
    mesh=plsc.VectorSubcoreMesh(
        core_axis_name="c", subcore_axis_name="s", num_cores=NC, num_subcores=NS
    ),
    scratch_types=(
        [pltpu.VMEM_SHARED((NUM_EDGE_EMB, DIM), jnp.float32)]
        + [pltpu.VMEM((CH2,), jnp.int32)] * NBUF
        + [pltpu.VMEM((CH,), jnp.int32)] * NBUF
        + [pltpu.VMEM((CH, DIM), jnp.float32)] * NBUF
        + [pltpu.SemaphoreType.DMA] * (3 * NBUF)
    ),
    compiler_params=pltpu.CompilerParams(needs_layout_passes=False),
)(_edge_gather_body)


LEGO_BLK = 1000
POINT_BLK = 5000
GRID = N_LEGO // LEGO_BLK


def _dense_body(ids_ref, lego_pos_ref, point_pos_ref, brick_ref, wpos_ref,
                bpos_ref, wpoint_ref, bpoint_ref, lego_out_ref, point_out_ref):
    ids = ids_ref[0, 0, :]
    onehot = (ids[:, None] == lax.broadcasted_iota(
        jnp.int32, (LEGO_BLK, NUM_BRICKS), 1)).astype(jnp.bfloat16)
    brick = jnp.dot(onehot, brick_ref[...], preferred_element_type=jnp.float32)
    proj = jnp.dot(lego_pos_ref[...], wpos_ref[...],
                   preferred_element_type=jnp.float32)
    lego_out_ref[...] = brick + proj + bpos_ref[...]
    point_out_ref[...] = jnp.dot(point_pos_ref[...], wpoint_ref[...],
                                 preferred_element_type=jnp.float32) + bpoint_ref[...]


def _dense(ids3, lego_pos, point_pos, brick_bf16, w_pos, b_pos, w_point, b_point):
    return pl.pallas_call(
        _dense_body,
        grid=(GRID,),
        in_specs=[
            pl.BlockSpec((1, 1, LEGO_BLK), lambda i: (i, 0, 0)),
            pl.BlockSpec((LEGO_BLK, 8), lambda i: (i, 0)),
            pl.BlockSpec((POINT_BLK, 8), lambda i: (i, 0)),
            pl.BlockSpec((NUM_BRICKS, DIM), lambda i: (0, 0)),
            pl.BlockSpec((8, DIM), lambda i: (0, 0)),
            pl.BlockSpec((1, DIM), lambda i: (0, 0)),
            pl.BlockSpec((8, DIM), lambda i: (0, 0)),
            pl.BlockSpec((1, DIM), lambda i: (0, 0)),
        ],
        out_specs=[
            pl.BlockSpec((LEGO_BLK, DIM), lambda i: (i, 0)),
            pl.BlockSpec((POINT_BLK, DIM), lambda i: (i, 0)),
        ],
        out_shape=[
            jax.ShapeDtypeStruct((N_LEGO, DIM), jnp.float32),
            jax.ShapeDtypeStruct((N_POINT, DIM), jnp.float32),
        ],
    )(ids3, lego_pos, point_pos, brick_bf16, w_pos, b_pos, w_point, b_point)


def kernel(lego_brick_ids, lego_pos, edge_shifts, point_pos,
           brick_table, edge_table, W_pos, b_pos, W_point, b_point):
    edge_attr = _edge_gather(
        edge_shifts.astype(jnp.int32).reshape(N_EDGE * 2), edge_table)

    ids3 = lego_brick_ids.astype(jnp.int32).reshape(GRID, 1, LEGO_BLK)
    lego_pos8 = jnp.pad(lego_pos, ((0, 0), (0, 2)))
    point_pos8 = jnp.pad(point_pos, ((0, 0), (0, 5)))
    wpos8 = jnp.pad(W_pos, ((0, 2), (0, 0)))
    wpoint8 = jnp.pad(W_point, ((0, 5), (0, 0)))
    lego_x, point_x = _dense(
        ids3, lego_pos8, point_pos8, brick_table.astype(jnp.bfloat16),
        wpos8, b_pos.reshape(1, DIM), wpoint8, b_point.reshape(1, DIM))
    return (lego_x, edge_attr, point_x)


# speedup vs baseline: 6.6873x; 2.5294x over previous
import functools

import jax
import jax.numpy as jnp
from jax import lax
from jax.experimental import pallas as pl
from jax.experimental.pallas import tpu as pltpu
from jax.experimental.pallas import tpu_sc as plsc

DIM = 128
NUM_X_SHIFTS = 15
NUM_EDGE_EMB = 225
NUM_BRICKS = 1024
N_LEGO = 10000
N_EDGE = 320000
N_POINT = 50000

NC = 2
NS = 16
NW = NC * NS
L = 16

CH = 128
NBLK = N_EDGE // CH
T_MAIN = NBLK // NW
REM = NBLK - T_MAIN * NW
NBUF = 4


def _edge_gather_body(s0_hbm, s1_hbm, table_hbm, out_hbm, table_v, *scratch):
    shifts_v = scratch[0:NBUF]
    idx_v = scratch[NBUF:2 * NBUF]
    rows_v = scratch[2 * NBUF:3 * NBUF]
    ssem = scratch[3 * NBUF:4 * NBUF]
    gsem = scratch[4 * NBUF:5 * NBUF]
    osem = scratch[5 * NBUF:6 * NBUF]
    wid = lax.axis_index("s") * NC + lax.axis_index("c")

    @pl.when(lax.axis_index("s") == 0)
    def _():
        pltpu.sync_copy(table_hbm, table_v)

    plsc.subcore_barrier()

    def start_shifts(g, b):
        pltpu.async_copy(s0_hbm.at[pl.ds(g * CH, CH)],
                         shifts_v[b].at[pl.ds(0, CH)], ssem[b])
        pltpu.async_copy(s1_hbm.at[pl.ds(g * CH, CH)],
                         shifts_v[b].at[pl.ds(CH, CH)], ssem[b])

    def wait_shifts(g, b):
        pltpu.make_async_copy(s0_hbm.at[pl.ds(g * CH, CH)],
                              shifts_v[b].at[pl.ds(0, CH)], ssem[b]).wait()
        pltpu.make_async_copy(s1_hbm.at[pl.ds(g * CH, CH)],
                              shifts_v[b].at[pl.ds(CH, CH)], ssem[b]).wait()

    def compute_idx(b):
        sv = shifts_v[b]
        iv = idx_v[b]
        for j in range(CH // L):
            s0 = sv[pl.ds(j * L, L)]
            s1 = sv[pl.ds(CH + j * L, L)]
            iv[pl.ds(j * L, L)] = s0 * NUM_X_SHIFTS + s1

    def start_gather(b):
        pltpu.async_copy(table_v.at[idx_v[b]], rows_v[b], gsem[b])

    def wait_gather(b):
        pltpu.make_async_copy(table_v.at[idx_v[b]], rows_v[b],
                              gsem[b]).wait()

    def start_out(g, b):
        pltpu.async_copy(rows_v[b], out_hbm.at[pl.ds(g * CH, CH)], osem[b])

    def wait_out(g, b):
        pltpu.make_async_copy(rows_v[b], out_hbm.at[pl.ds(g * CH, CH)],
                              osem[b]).wait()

    def g_of(t):
        return t * NW + wid

    for b in range(NBUF):
        start_shifts(g_of(b), b)
    for t in range(NBUF):
        b = t % NBUF
        wait_shifts(g_of(t), b)
        compute_idx(b)
        start_gather(b)
        start_shifts(g_of(t + NBUF), b)
        if t >= 1:
            wait_gather((t - 1) % NBUF)
            start_out(g_of(t - 1), (t - 1) % NBUF)

    def steady(k, carry):
        for b in range(NBUF):
            t = k * NBUF + b
            wait_shifts(g_of(t), b)
            compute_idx(b)
            wait_out(g_of(t - NBUF), b)
            start_gather(b)
            start_shifts(g_of(t + NBUF), b)
            b1 = (b - 1) % NBUF
            wait_gather(b1)
            start_out(g_of(t - 1), b1)
        return carry

    lax.fori_loop(1, T_MAIN // NBUF - 1, steady, 0)

    for t in range(T_MAIN - 6, T_MAIN):
        b = t % NBUF
        wait_shifts(g_of(t), b)
        compute_idx(b)
        wait_out(g_of(t - NBUF), b)
        start_gather(b)
        if t + NBUF < T_MAIN:
            start_shifts(g_of(t + NBUF), b)
        b1 = (b - 1) % NBUF
        wait_gather(b1)
        start_out(g_of(t - 1), b1)
    bl = (T_MAIN - 1) % NBUF
    wait_gather(bl)
    start_out(g_of(T_MAIN - 1), bl)
    for t in range(T_MAIN - NBUF, T_MAIN):
        wait_out(g_of(t), t % NBUF)

    @pl.when(wid < REM)
    def _():
        g = T_MAIN * NW + wid
        pltpu.sync_copy(s0_hbm.at[pl.ds(g * CH, CH)],
                        shifts_v[0].at[pl.ds(0, CH)])
        pltpu.sync_copy(s1_hbm.at[pl.ds(g * CH, CH)],
                        shifts_v[0].at[pl.ds(CH, CH)])
        compute_idx(0)
        pltpu.async_copy(table_v.at[idx_v[0]], rows_v[0], gsem[0]).wait()
        pltpu.sync_copy(rows_v[0], out_hbm.at[pl.ds(g * CH, CH)])


_edge_gather = functools.partial(
    pl.kernel,
    out_type=jax.ShapeDtypeStruct((N_EDGE, DIM), jnp.float32),
    mesh=plsc.VectorSubcoreMesh(
        core_axis_name="c", subcore_axis_name="s", num_cores=NC, num_subcores=NS
    ),
    scratch_types=(
        [pltpu.VMEM_SHARED((NUM_EDGE_EMB, DIM), jnp.float32)]
        + [pltpu.VMEM((2 * CH,), jnp.int32)] * NBUF
        + [pltpu.VMEM((CH,), jnp.int32)] * NBUF
        + [pltpu.VMEM((CH, DIM), jnp.float32)] * NBUF
        + [pltpu.SemaphoreType.DMA] * (3 * NBUF)
    ),
    compiler_params=pltpu.CompilerParams(needs_layout_passes=False),
)(_edge_gather_body)


LEGO_BLK = 1000
POINT_BLK = 5000
GRID = N_LEGO // LEGO_BLK


def _dense_body(ids_ref, lego_pos_ref, point_pos_ref, brick_ref, wpos_ref,
                bpos_ref, wpoint_ref, bpoint_ref, lego_out_ref, point_out_ref):
    ids = ids_ref[0, 0, :]
    onehot = (ids[:, None] == lax.broadcasted_iota(
        jnp.int32, (LEGO_BLK, NUM_BRICKS), 1)).astype(jnp.bfloat16)
    brick = jnp.dot(onehot, brick_ref[...], preferred_element_type=jnp.float32)
    proj = jnp.dot(lego_pos_ref[...], wpos_ref[...],
                   preferred_element_type=jnp.float32)
    lego_out_ref[...] = brick + proj + bpos_ref[...]
    point_out_ref[...] = jnp.dot(point_pos_ref[...], wpoint_ref[...],
                                 preferred_element_type=jnp.float32) + bpoint_ref[...]


def _dense(ids3, lego_pos, point_pos, brick_bf16, w_pos, b_pos, w_point, b_point):
    return pl.pallas_call(
        _dense_body,
        grid=(GRID,),
        in_specs=[
            pl.BlockSpec((1, 1, LEGO_BLK), lambda i: (i, 0, 0)),
            pl.BlockSpec((LEGO_BLK, 6), lambda i: (i, 0)),
            pl.BlockSpec((POINT_BLK, 3), lambda i: (i, 0)),
            pl.BlockSpec((NUM_BRICKS, DIM), lambda i: (0, 0)),
            pl.BlockSpec((6, DIM), lambda i: (0, 0)),
            pl.BlockSpec((1, DIM), lambda i: (0, 0)),
            pl.BlockSpec((3, DIM), lambda i: (0, 0)),
            pl.BlockSpec((1, DIM), lambda i: (0, 0)),
        ],
        out_specs=[
            pl.BlockSpec((LEGO_BLK, DIM), lambda i: (i, 0)),
            pl.BlockSpec((POINT_BLK, DIM), lambda i: (i, 0)),
        ],
        out_shape=[
            jax.ShapeDtypeStruct((N_LEGO, DIM), jnp.float32),
            jax.ShapeDtypeStruct((N_POINT, DIM), jnp.float32),
        ],
    )(ids3, lego_pos, point_pos, brick_bf16, w_pos, b_pos, w_point, b_point)


def kernel(lego_brick_ids, lego_pos, edge_shifts, point_pos,
           brick_table, edge_table, W_pos, b_pos, W_point, b_point):
    shifts32 = edge_shifts.astype(jnp.int32)
    edge_attr = _edge_gather(shifts32[:, 0], shifts32[:, 1], edge_table)

    ids3 = lego_brick_ids.astype(jnp.int32).reshape(GRID, 1, LEGO_BLK)
    lego_x, point_x = _dense(
        ids3, lego_pos, point_pos, brick_table.astype(jnp.bfloat16),
        W_pos, b_pos.reshape(1, DIM), W_point, b_point.reshape(1, DIM))
    return (lego_x, edge_attr, point_x)


# speedup vs baseline: 6.7155x; 1.0042x over previous
import functools

import jax
import jax.numpy as jnp
from jax import lax
from jax.experimental import pallas as pl
from jax.experimental.pallas import tpu as pltpu
from jax.experimental.pallas import tpu_sc as plsc

DIM = 128
NUM_X_SHIFTS = 15
NUM_EDGE_EMB = 225
NUM_BRICKS = 1024
N_LEGO = 10000
N_EDGE = 320000
N_POINT = 50000

NC = 2
NS = 16
NW = NC * NS
L = 16

CH = 128
NBLK = N_EDGE // CH
T_MAIN = NBLK // NW
REM = NBLK - T_MAIN * NW
NBUF = 4


def _edge_gather_body(idx_hbm, table_hbm, out_hbm, table_v, *scratch):
    idx_v = scratch[0:NBUF]
    rows_v = scratch[NBUF:2 * NBUF]
    ssem = scratch[2 * NBUF:3 * NBUF]
    gsem = scratch[3 * NBUF:4 * NBUF]
    osem = scratch[4 * NBUF:5 * NBUF]
    wid = lax.axis_index("s") * NC + lax.axis_index("c")

    @pl.when(lax.axis_index("s") == 0)
    def _():
        pltpu.sync_copy(table_hbm, table_v)

    plsc.subcore_barrier()

    def start_idx(g, b):
        pltpu.async_copy(idx_hbm.at[pl.ds(g * CH, CH)], idx_v[b], ssem[b])

    def wait_idx(g, b):
        pltpu.make_async_copy(idx_hbm.at[pl.ds(g * CH, CH)], idx_v[b],
                              ssem[b]).wait()

    def start_gather(b):
        pltpu.async_copy(table_v.at[idx_v[b]], rows_v[b], gsem[b])

    def wait_gather(b):
        pltpu.make_async_copy(table_v.at[idx_v[b]], rows_v[b],
                              gsem[b]).wait()

    def start_out(g, b):
        pltpu.async_copy(rows_v[b], out_hbm.at[pl.ds(g * CH, CH)], osem[b])

    def wait_out(g, b):
        pltpu.make_async_copy(rows_v[b], out_hbm.at[pl.ds(g * CH, CH)],
                              osem[b]).wait()

    def g_of(t):
        return t * NW + wid

    def tail(t, b1, prefetch=True):
        wait_gather(b1)
        start_out(g_of(t - 1), b1)
        if prefetch:
            start_idx(g_of(t + NBUF - 1), b1)

    for b in range(NBUF):
        start_idx(g_of(b), b)
    for t in range(NBUF):
        b = t % NBUF
        wait_idx(g_of(t), b)
        start_gather(b)
        if t >= 1:
            tail(t, (t - 1) % NBUF)

    def steady(k, carry):
        for b in range(NBUF):
            t = k * NBUF + b
            wait_idx(g_of(t), b)
            wait_out(g_of(t - NBUF), b)
            start_gather(b)
            tail(t, (b - 1) % NBUF)
        return carry

    lax.fori_loop(1, T_MAIN // NBUF - 1, steady, 0)

    for t in range(T_MAIN - 6, T_MAIN):
        b = t % NBUF
        wait_idx(g_of(t), b)
        wait_out(g_of(t - NBUF), b)
        start_gather(b)
        tail(t, (t - 1) % NBUF, prefetch=(t + NBUF - 1 < T_MAIN))
    bl = (T_MAIN - 1) % NBUF
    wait_gather(bl)
    start_out(g_of(T_MAIN - 1), bl)
    for t in range(T_MAIN - NBUF, T_MAIN):
        wait_out(g_of(t), t % NBUF)

    @pl.when(wid < REM)
    def _():
        g = T_MAIN * NW + wid
        pltpu.sync_copy(idx_hbm.at[pl.ds(g * CH, CH)], idx_v[0])
        pltpu.async_copy(table_v.at[idx_v[0]], rows_v[0], gsem[0]).wait()
        pltpu.sync_copy(rows_v[0], out_hbm.at[pl.ds(g * CH, CH)])


_edge_gather = functools.partial(
    pl.kernel,
    out_type=jax.ShapeDtypeStruct((N_EDGE, DIM), jnp.float32),
    mesh=plsc.VectorSubcoreMesh(
        core_axis_name="c", subcore_axis_name="s", num_cores=NC, num_subcores=NS
    ),
    scratch_types=(
        [pltpu.VMEM_SHARED((NUM_EDGE_EMB, DIM), jnp.float32)]
        + [pltpu.VMEM((CH,), jnp.int32)] * NBUF
        + [pltpu.VMEM((CH, DIM), jnp.float32)] * NBUF
        + [pltpu.SemaphoreType.DMA] * (3 * NBUF)
    ),
    compiler_params=pltpu.CompilerParams(needs_layout_passes=False),
)(_edge_gather_body)


LEGO_BLK = 1000
POINT_BLK = 5000
GRID = N_LEGO // LEGO_BLK


def _dense_body(ids_ref, lego_pos_ref, point_pos_ref, brick_ref, wpos_ref,
                bpos_ref, wpoint_ref, bpoint_ref, lego_out_ref, point_out_ref):
    ids = ids_ref[0, 0, :]
    onehot = (ids[:, None] == lax.broadcasted_iota(
        jnp.int32, (LEGO_BLK, NUM_BRICKS), 1)).astype(jnp.bfloat16)
    brick = jnp.dot(onehot, brick_ref[...], preferred_element_type=jnp.float32)
    proj = jnp.dot(lego_pos_ref[...], wpos_ref[...],
                   preferred_element_type=jnp.float32)
    lego_out_ref[...] = brick + proj + bpos_ref[...]
    point_out_ref[...] = jnp.dot(point_pos_ref[...], wpoint_ref[...],
                                 preferred_element_type=jnp.float32) + bpoint_ref[...]


def _dense(ids3, lego_pos, point_pos, brick_bf16, w_pos, b_pos, w_point, b_point):
    return pl.pallas_call(
        _dense_body,
        grid=(GRID,),
        in_specs=[
            pl.BlockSpec((1, 1, LEGO_BLK), lambda i: (i, 0, 0)),
            pl.BlockSpec((LEGO_BLK, 6), lambda i: (i, 0)),
            pl.BlockSpec((POINT_BLK, 3), lambda i: (i, 0)),
            pl.BlockSpec((NUM_BRICKS, DIM), lambda i: (0, 0)),
            pl.BlockSpec((6, DIM), lambda i: (0, 0)),
            pl.BlockSpec((1, DIM), lambda i: (0, 0)),
            pl.BlockSpec((3, DIM), lambda i: (0, 0)),
            pl.BlockSpec((1, DIM), lambda i: (0, 0)),
        ],
        out_specs=[
            pl.BlockSpec((LEGO_BLK, DIM), lambda i: (i, 0)),
            pl.BlockSpec((POINT_BLK, DIM), lambda i: (i, 0)),
        ],
        out_shape=[
            jax.ShapeDtypeStruct((N_LEGO, DIM), jnp.float32),
            jax.ShapeDtypeStruct((N_POINT, DIM), jnp.float32),
        ],
    )(ids3, lego_pos, point_pos, brick_bf16, w_pos, b_pos, w_point, b_point)


def kernel(lego_brick_ids, lego_pos, edge_shifts, point_pos,
           brick_table, edge_table, W_pos, b_pos, W_point, b_point):
    shifts32 = edge_shifts.astype(jnp.int32)
    idx32 = shifts32[:, 0] * NUM_X_SHIFTS + shifts32[:, 1]
    edge_attr = _edge_gather(idx32, edge_table)

    ids3 = lego_brick_ids.astype(jnp.int32).reshape(GRID, 1, LEGO_BLK)
    lego_x, point_x = _dense(
        ids3, lego_pos, point_pos, brick_table.astype(jnp.bfloat16),
        W_pos, b_pos.reshape(1, DIM), W_point, b_point.reshape(1, DIM))
    return (lego_x, edge_attr, point_x)


# speedup vs baseline: 6.7269x; 1.0017x over previous
import functools

import jax
import jax.numpy as jnp
from jax import lax
from jax.experimental import pallas as pl
from jax.experimental.pallas import tpu as pltpu
from jax.experimental.pallas import tpu_sc as plsc

DIM = 128
NUM_X_SHIFTS = 15
NUM_EDGE_EMB = 225
NUM_BRICKS = 1024
N_LEGO = 10000
N_EDGE = 320000
N_POINT = 50000

NC = 2
NS = 16
NW = NC * NS
L = 16

CH = 128
NBLK = N_EDGE // CH
T_MAIN = NBLK // NW
REM = NBLK - T_MAIN * NW
NBUF = 4


def _edge_gather_body(idx_hbm, table_hbm, out_hbm, table_v, *scratch):
    idx_v = scratch[0:NBUF]
    rows_v = scratch[NBUF:2 * NBUF]
    ssem = scratch[2 * NBUF:3 * NBUF]
    gsem = scratch[3 * NBUF:4 * NBUF]
    osem = scratch[4 * NBUF:5 * NBUF]
    wid = lax.axis_index("s") * NC + lax.axis_index("c")
    sid = lax.axis_index("s")

    def start_idx(g, b):
        pltpu.async_copy(idx_hbm.at[pl.ds(g * CH, CH)], idx_v[b], ssem[b])

    def wait_idx(g, b):
        pltpu.make_async_copy(idx_hbm.at[pl.ds(g * CH, CH)], idx_v[b],
                              ssem[b]).wait()

    def g_of0(t):
        return t * NW + wid

    for b in range(NBUF):
        start_idx(g_of0(b), b)

    TROWS = 16
    @pl.when(sid < NUM_EDGE_EMB // TROWS)
    def _():
        pltpu.sync_copy(table_hbm.at[pl.ds(sid * TROWS, TROWS)],
                        table_v.at[pl.ds(sid * TROWS, TROWS)])

    @pl.when(sid == NUM_EDGE_EMB // TROWS)
    def _():
        rem_base = (NUM_EDGE_EMB // TROWS) * TROWS
        pltpu.sync_copy(table_hbm.at[pl.ds(rem_base, NUM_EDGE_EMB - rem_base)],
                        table_v.at[pl.ds(rem_base, NUM_EDGE_EMB - rem_base)])

    plsc.subcore_barrier()

    def start_gather(b):
        pltpu.async_copy(table_v.at[idx_v[b]], rows_v[b], gsem[b])

    def wait_gather(b):
        pltpu.make_async_copy(table_v.at[idx_v[b]], rows_v[b],
                              gsem[b]).wait()

    def start_out(g, b):
        pltpu.async_copy(rows_v[b], out_hbm.at[pl.ds(g * CH, CH)], osem[b])

    def wait_out(g, b):
        pltpu.make_async_copy(rows_v[b], out_hbm.at[pl.ds(g * CH, CH)],
                              osem[b]).wait()

    def g_of(t):
        return t * NW + wid

    def tail(t, b1, prefetch=True):
        wait_gather(b1)
        start_out(g_of(t - 1), b1)
        if prefetch:
            start_idx(g_of(t + NBUF - 1), b1)

    for t in range(NBUF):
        b = t % NBUF
        wait_idx(g_of(t), b)
        start_gather(b)
        if t >= 1:
            tail(t, (t - 1) % NBUF)

    def steady(k, carry):
        for b in range(NBUF):
            t = k * NBUF + b
            wait_idx(g_of(t), b)
            wait_out(g_of(t - NBUF), b)
            start_gather(b)
            tail(t, (b - 1) % NBUF)
        return carry

    lax.fori_loop(1, T_MAIN // NBUF - 1, steady, 0)

    for t in range(T_MAIN - 6, T_MAIN):
        b = t % NBUF
        wait_idx(g_of(t), b)
        wait_out(g_of(t - NBUF), b)
        start_gather(b)
        tail(t, (t - 1) % NBUF, prefetch=(t + NBUF - 1 < T_MAIN))
    bl = (T_MAIN - 1) % NBUF
    wait_gather(bl)
    start_out(g_of(T_MAIN - 1), bl)
    for t in range(T_MAIN - NBUF, T_MAIN):
        wait_out(g_of(t), t % NBUF)

    @pl.when(wid < REM)
    def _():
        g = T_MAIN * NW + wid
        pltpu.sync_copy(idx_hbm.at[pl.ds(g * CH, CH)], idx_v[0])
        pltpu.async_copy(table_v.at[idx_v[0]], rows_v[0], gsem[0]).wait()
        pltpu.sync_copy(rows_v[0], out_hbm.at[pl.ds(g * CH, CH)])


_edge_gather = functools.partial(
    pl.kernel,
    out_type=jax.ShapeDtypeStruct((N_EDGE, DIM), jnp.float32),
    mesh=plsc.VectorSubcoreMesh(
        core_axis_name="c", subcore_axis_name="s", num_cores=NC, num_subcores=NS
    ),
    scratch_types=(
        [pltpu.VMEM_SHARED((NUM_EDGE_EMB, DIM), jnp.float32)]
        + [pltpu.VMEM((CH,), jnp.int32)] * NBUF
        + [pltpu.VMEM((CH, DIM), jnp.float32)] * NBUF
        + [pltpu.SemaphoreType.DMA] * (3 * NBUF)
    ),
    compiler_params=pltpu.CompilerParams(needs_layout_passes=False),
)(_edge_gather_body)


LEGO_BLK = 1000
POINT_BLK = 5000
GRID = N_LEGO // LEGO_BLK


def _dense_body(ids_ref, lego_pos_ref, point_pos_ref, brick_ref, wpos_ref,
                bpos_ref, wpoint_ref, bpoint_ref, lego_out_ref, point_out_ref):
    ids = ids_ref[0, 0, :]
    onehot = (ids[:, None] == lax.broadcasted_iota(
        jnp.int32, (LEGO_BLK, NUM_BRICKS), 1)).astype(jnp.bfloat16)
    brick = jnp.dot(onehot, brick_ref[...], preferred_element_type=jnp.float32)
    proj = jnp.dot(lego_pos_ref[...], wpos_ref[...],
                   preferred_element_type=jnp.float32)
    lego_out_ref[...] = brick + proj + bpos_ref[...]
    point_out_ref[...] = jnp.dot(point_pos_ref[...], wpoint_ref[...],
                                 preferred_element_type=jnp.float32) + bpoint_ref[...]


def _dense(ids3, lego_pos, point_pos, brick_bf16, w_pos, b_pos, w_point, b_point):
    return pl.pallas_call(
        _dense_body,
        grid=(GRID,),
        in_specs=[
            pl.BlockSpec((1, 1, LEGO_BLK), lambda i: (i, 0, 0)),
            pl.BlockSpec((LEGO_BLK, 6), lambda i: (i, 0)),
            pl.BlockSpec((POINT_BLK, 3), lambda i: (i, 0)),
            pl.BlockSpec((NUM_BRICKS, DIM), lambda i: (0, 0)),
            pl.BlockSpec((6, DIM), lambda i: (0, 0)),
            pl.BlockSpec((1, DIM), lambda i: (0, 0)),
            pl.BlockSpec((3, DIM), lambda i: (0, 0)),
            pl.BlockSpec((1, DIM), lambda i: (0, 0)),
        ],
        out_specs=[
            pl.BlockSpec((LEGO_BLK, DIM), lambda i: (i, 0)),
            pl.BlockSpec((POINT_BLK, DIM), lambda i: (i, 0)),
        ],
        out_shape=[
            jax.ShapeDtypeStruct((N_LEGO, DIM), jnp.float32),
            jax.ShapeDtypeStruct((N_POINT, DIM), jnp.float32),
        ],
    )(ids3, lego_pos, point_pos, brick_bf16, w_pos, b_pos, w_point, b_point)


def kernel(lego_brick_ids, lego_pos, edge_shifts, point_pos,
           brick_table, edge_table, W_pos, b_pos, W_point, b_point):
    shifts32 = edge_shifts.astype(jnp.int32)
    idx32 = shifts32[:, 0] * NUM_X_SHIFTS + shifts32[:, 1]
    edge_attr = _edge_gather(idx32, edge_table)

    ids3 = lego_brick_ids.astype(jnp.int32).reshape(GRID, 1, LEGO_BLK)
    lego_x, point_x = _dense(
        ids3, lego_pos, point_pos, brick_table.astype(jnp.bfloat16),
        W_pos, b_pos.reshape(1, DIM), W_point, b_point.reshape(1, DIM))
    return (lego_x, edge_attr, point_x)
